# baseline (speedup 1.0000x reference)
import functools

import jax
import jax.numpy as jnp
from jax import lax
from jax.experimental import pallas as pl
from jax.experimental.pallas import tpu as pltpu
from jax.experimental.pallas import tpu_sc as plsc

_K = 4
_V = 100000
_D = 64
_B = 4096
_S = 200
_J = _S // _K
_N = _B * _S
_NW = 32
_PER_W = _N // _NW
_BPW = _B // _NW
_BLK = _K * _BPW


def _make_kernel():
    mesh = plsc.VectorSubcoreMesh(core_axis_name="c", subcore_axis_name="s")

    @functools.partial(
        pl.kernel,
        mesh=mesh,
        out_type=jax.ShapeDtypeStruct((_J * 8, _NW, 1024), jnp.float32),
        compiler_params=pltpu.CompilerParams(
            use_tc_tiling_on_sc=False, needs_layout_passes=False
        ),
        scratch_types=[
            pltpu.VMEM((_PER_W,), jnp.int32),
            pltpu.VMEM((_BLK,), jnp.int32),
            pltpu.VMEM((2, 4, 128), jnp.int32),
            pltpu.VMEM((2, _BLK, _D), jnp.float32),
            pltpu.VMEM((2, 8, 1024), jnp.float32),
            pltpu.SemaphoreType.DMA,
            pltpu.SemaphoreType.DMA,
            pltpu.SemaphoreType.DMA,
            pltpu.SemaphoreType.DMA,
        ],
    )
    def body(ids_hbm, table_hbm, out_hbm, idx_v, pos_v, sidx_v, rows_v, ot_v,
             sg0, sg1, so0, so1):
        wid = lax.axis_index("c") * 16 + lax.axis_index("s")
        sgs = (sg0, sg1)
        sos = (so0, so1)

        pltpu.sync_copy(ids_hbm.at[pl.ds(wid * _PER_W, _PER_W)], idx_v)

        iota = lax.iota(jnp.int32, 16)
        offv = (iota & 3) * _V
        for ch in range(_BLK // 16):
            base = ch * 16
            pos = ((base // 4) + lax.shift_right_logical(iota, 2)) * _S + (iota & 3)
            pos_v[pl.ds(base, 16)] = pos

        def build_list(j, buf):
            for ch in range(_BLK // 16):
                sl = pl.ds((ch % 8) * 16, 16)
                pos = pos_v[pl.ds(ch * 16, 16)] + 4 * j
                ids16 = plsc.load_gather(idx_v, [pos])
                sidx_v[buf, ch // 8, sl] = ids16 + offv

        def fire_gathers(j, buf):
            build_list(j, buf)
            for s in range(4):
                pltpu.async_copy(
                    table_hbm.at[sidx_v.at[buf].at[s]],
                    rows_v.at[buf].at[pl.ds(s * 128, 128)],
                    sgs[buf],
                )

        def wait_gathers(buf):
            for s in range(4):
                pltpu.make_async_copy(
                    table_hbm.at[sidx_v.at[buf].at[s]],
                    rows_v.at[buf].at[pl.ds(s * 128, 128)],
                    sgs[buf],
                ).wait()

        def out_slice(j):
            return out_hbm.at[pl.ds(j * 8, 8), wid]

        fire_gathers(0, 0)

        def outer(jj, carry):
            for bf in range(2):
                j = jj * 2 + bf

                @pl.when(j + 1 < _J)
                def _():
                    fire_gathers(j + 1, 1 - bf)

                wait_gathers(bf)

                @pl.when(j >= 2)
                def _():
                    pltpu.make_async_copy(ot_v.at[bf], out_slice(j - 2), sos[bf]).wait()

                rv = rows_v.at[bf]
                ot = ot_v.at[bf]

                @plsc.parallel_loop(0, _BPW, unroll=2)
                def _(b):
                    r = b * 4
                    colv = lax.shift_left((iota & 7), 7) + b
                    for c in range(4):
                        sl = pl.ds(c * 16, 16)
                        acc = rv[r, sl] + rv[r + 1, sl] + rv[r + 2, sl] + rv[r + 3, sl]
                        plsc.store_scatter(ot, [c * 2 + lax.shift_right_logical(iota, 3), colv], acc)

                pltpu.async_copy(ot_v.at[bf], out_slice(j), sos[bf])
            return carry

        lax.fori_loop(0, _J // 2, outer, 0)

        for bf in range(2):
            pltpu.make_async_copy(ot_v.at[bf], out_slice(_J - 2 + bf), sos[bf]).wait()

    return body


_sc_kernel = _make_kernel()


@jax.jit
def kernel(input_ids, tables):
    ids_flat = input_ids.reshape(_N)
    table_flat = tables.reshape(_K * _V, _D)
    out4 = _sc_kernel(ids_flat, table_flat)
    z = out4.reshape(_J, 8, _NW, 8, 128).transpose(2, 4, 0, 1, 3)
    return z.reshape(_B, _J, _D)


# speedup vs baseline: 1.2538x; 1.2538x over previous
import functools

import jax
import jax.numpy as jnp
from jax import lax
from jax.experimental import pallas as pl
from jax.experimental.pallas import tpu as pltpu
from jax.experimental.pallas import tpu_sc as plsc

_K = 4
_V = 100000
_D = 64
_B = 4096
_S = 200
_N = _B * _S
_NW = 32
_PER_W = _N // _NW
_IDX_ROWS = _PER_W // 128
_BLK = 512
_NBLK = _PER_W // _BLK
_OUT_BLK = _BLK // _K
_OUT_ROWS = _N // _K


def _make_kernel():
    mesh = plsc.VectorSubcoreMesh(core_axis_name="c", subcore_axis_name="s")

    @functools.partial(
        pl.kernel,
        mesh=mesh,
        out_type=jax.ShapeDtypeStruct((_OUT_ROWS, _D), jnp.float32),
        compiler_params=pltpu.CompilerParams(use_tc_tiling_on_sc=False),
        scratch_types=[
            pltpu.VMEM((_IDX_ROWS, 128), jnp.int32),
            pltpu.VMEM((2, _BLK, _D), jnp.float32),
            pltpu.VMEM((2, _OUT_BLK, _D), jnp.float32),
            pltpu.SemaphoreType.DMA,
            pltpu.SemaphoreType.DMA,
            pltpu.SemaphoreType.DMA,
            pltpu.SemaphoreType.DMA,
        ],
    )
    def body(ids_hbm, table_hbm, out_hbm, idx_v, rows_v, out_v, sg0, sg1, so0, so1):
        wid = lax.axis_index("c") * 16 + lax.axis_index("s")
        sgs = (sg0, sg1)
        sos = (so0, so1)

        pltpu.sync_copy(ids_hbm.at[pl.ds(wid * _IDX_ROWS, _IDX_ROWS)], idx_v)
        iota = lax.iota(jnp.int32, 16)
        offv = ((iota % _K) - (iota % 2)) * _V + (iota % 2)

        @plsc.parallel_loop(0, _IDX_ROWS, unroll=2)
        def _(r):
            for c in range(8):
                sl = pl.ds(c * 16, 16)
                idx_v[r, sl] = idx_v[r, sl] * 2 + offv

        def fire_gathers(g, buf):
            for s in range(4):
                pltpu.async_copy(
                    table_hbm.at[idx_v.at[g * 4 + s]],
                    rows_v.at[buf].at[pl.ds(s * 128, 128)],
                    sgs[buf],
                )

        def wait_gathers(g, buf):
            for s in range(4):
                pltpu.make_async_copy(
                    table_hbm.at[idx_v.at[g * 4 + s]],
                    rows_v.at[buf].at[pl.ds(s * 128, 128)],
                    sgs[buf],
                ).wait()

        def out_slice(g):
            return out_hbm.at[pl.ds(wid * (_PER_W // _K) + g * _OUT_BLK, _OUT_BLK)]

        fire_gathers(0, 0)

        def outer(gg, carry):
            for b in range(2):
                g = gg * 2 + b

                @pl.when(g + 1 < _NBLK)
                def _():
                    fire_gathers(g + 1, 1 - b)

                wait_gathers(g, b)

                @pl.when(g >= 2)
                def _():
                    pltpu.make_async_copy(out_v.at[b], out_slice(g - 2), sos[b]).wait()

                rv = rows_v.at[b]
                ov = out_v.at[b]

                @plsc.parallel_loop(0, _OUT_BLK, unroll=4)
                def _(j):
                    r = j * 4
                    for c in range(4):
                        sl = pl.ds(c * 16, 16)
                        ov[j, sl] = (
                            rv[r, sl] + rv[r + 1, sl] + rv[r + 2, sl] + rv[r + 3, sl]
                        )

                pltpu.async_copy(out_v.at[b], out_slice(g), sos[b])
            return carry

        lax.fori_loop(0, _NBLK // 2, outer, 0)

        for b in range(2):
            pltpu.make_async_copy(out_v.at[b], out_slice(_NBLK - 2 + b), sos[b]).wait()

    return body


_sc_kernel = _make_kernel()

_VC = 2048
_NG = (_V + _VC - 1) // _VC


def _tc_format_body(in_ref, out_ref):
    y0 = jnp.swapaxes(in_ref[0], 0, 1)
    y1 = jnp.swapaxes(in_ref[1], 0, 1)
    out_ref[0] = jnp.concatenate([y0, y1], axis=1)


def _format_table(tables):
    t = jnp.swapaxes(tables, 1, 2)
    out = pl.pallas_call(
        _tc_format_body,
        grid=(2, _NG),
        in_specs=[pl.BlockSpec((2, _D, _VC), lambda p, g: (p, 0, g))],
        out_specs=pl.BlockSpec((1, _VC, 128), lambda p, g: (p, g, 0)),
        out_shape=jax.ShapeDtypeStruct((2, _V, 128), jnp.float32),
    )(t)
    return out.reshape(_K * _V, _D)


@jax.jit
def kernel(input_ids, tables):
    ids2d = input_ids.reshape(_N // 128, 128)
    table_flat = _format_table(tables)
    out = _sc_kernel(ids2d, table_flat)
    return out.reshape(_B, _S // _K, _D)
